# Initial kernel scaffold; baseline (speedup 1.0000x reference)
import functools

import jax
import jax.numpy as jnp
from jax import lax
from jax.experimental import pallas as pl
from jax.experimental.pallas import tpu as pltpu
from jax.experimental.pallas import tpu_sc as plsc

N = 10000
E = 320000
D = 128
C = 128
OUT = 16

NW = 32
CHUNK = 128
NCHUNK = 80
EPT = CHUNK * NCHUNK
EPAD = EPT * NW
R = 10240
RPT = R // 16
TRASH = N

NID_PAD = 10240
GCHUNK = 80
GPT = NID_PAD // NW


def _mesh():
    return plsc.VectorSubcoreMesh(core_axis_name="c", subcore_axis_name="s")


@functools.lru_cache(maxsize=None)
def _edge_pass(width):
    @functools.partial(
        pl.kernel,
        out_type=jax.ShapeDtypeStruct((2, R, width), jnp.float32),
        mesh=_mesh(),
        scratch_types=[
            pltpu.VMEM((NCHUNK, CHUNK), jnp.int32),
            pltpu.VMEM((NCHUNK, CHUNK), jnp.int32),
            pltpu.VMEM((2 * CHUNK, width), jnp.float32),
            pltpu.VMEM_SHARED((R, width), jnp.float32),
            pltpu.SemaphoreType.DMA,
        ],
        name=f"sage_edge_pass_{width}",
    )
    def kern(table, src3, dst3, zrow, parts, src_v, dst_v, rows_v, acc, gsem):
        c = lax.axis_index("c")
        s = lax.axis_index("s")
        wid = s * 2 + c

        pltpu.sync_copy(zrow, rows_v.at[pl.ds(0, CHUNK)])
        for k in range(RPT // CHUNK):
            pltpu.sync_copy(rows_v.at[pl.ds(0, CHUNK)],
                            acc.at[pl.ds(s * RPT + k * CHUNK, CHUNK)])
        plsc.subcore_barrier()

        pltpu.sync_copy(src3.at[wid], src_v)
        pltpu.sync_copy(dst3.at[wid], dst_v)

        pltpu.async_copy(table.at[src_v.at[0]], rows_v.at[pl.ds(0, CHUNK)], gsem)

        def body(j, carry):
            b = lax.rem(j, 2)
            off = b * CHUNK
            pltpu.make_async_copy(
                table.at[src_v.at[0]], rows_v.at[pl.ds(0, CHUNK)], gsem).wait()

            @pl.when(j < NCHUNK - 1)
            def _():
                pltpu.async_copy(table.at[src_v.at[j + 1]],
                                 rows_v.at[pl.ds((1 - b) * CHUNK, CHUNK)], gsem)

            pltpu.sync_copy(rows_v.at[pl.ds(off, CHUNK)],
                            acc.at[dst_v.at[j]], add=True)
            return carry

        lax.fori_loop(0, NCHUNK, body, 0)
        plsc.subcore_barrier()

        for k in range(RPT // CHUNK):
            base = s * RPT + k * CHUNK
            pltpu.sync_copy(acc.at[pl.ds(base, CHUNK)],
                            rows_v.at[pl.ds(0, CHUNK)])
            pltpu.sync_copy(rows_v.at[pl.ds(0, CHUNK)],
                            parts.at[c, pl.ds(base, CHUNK)])

    return kern


@functools.lru_cache(maxsize=None)
def _emb_gather():
    @functools.partial(
        pl.kernel,
        out_type=jax.ShapeDtypeStruct((NID_PAD, C), jnp.float32),
        mesh=_mesh(),
        scratch_types=[
            pltpu.VMEM((GPT // GCHUNK, GCHUNK), jnp.int32),
            pltpu.VMEM((GCHUNK, C), jnp.float32),
            pltpu.SemaphoreType.DMA,
        ],
        name="emb_gather",
    )
    def kern(table, idx3, out, idx_v, rows_v, gsem):
        c = lax.axis_index("c")
        s = lax.axis_index("s")
        wid = s * 2 + c
        base = wid * GPT
        pltpu.sync_copy(idx3.at[wid], idx_v)
        for j in range(GPT // GCHUNK):
            pltpu.async_copy(table.at[idx_v.at[j]], rows_v, gsem).wait()
            pltpu.sync_copy(rows_v, out.at[pl.ds(base + j * GCHUNK, GCHUNK)])

    return kern


BM = 2000
GRID = N // BM


def _row_spec(w):
    return pl.BlockSpec((BM, w), lambda i: (i, 0))


def _full_spec(a, b):
    return pl.BlockSpec((a, b), lambda i: (0, 0))


def _tc_call(body, in_specs, out_specs, out_shapes):
    return pl.pallas_call(
        body,
        grid=(GRID,),
        in_specs=in_specs,
        out_specs=out_specs,
        out_shape=out_shapes,
    )


def _encode_body(x_ref, t_ref, er_ref, w_ref, b_ref, wt_ref, o_ref):
    h = jnp.dot(x_ref[...], w_ref[...], preferred_element_type=jnp.float32)
    h = h + b_ref[...] + t_ref[...] * wt_ref[...] + er_ref[...]
    ones_col = (lax.broadcasted_iota(jnp.int32, (BM, 16), 1) == 0)
    o_ref[...] = jnp.concatenate([h, ones_col.astype(jnp.float32)], axis=-1)


def _sage1_body(h_ref, p0_ref, p1_ref, ws_ref, wn_ref, h1_ref, dinv_ref):
    p = p0_ref[...] + p1_ref[...]
    dinv = 1.0 / jnp.maximum(p[:, 128:129], 1.0)
    mean = p[:, :128] * dinv
    h1 = jnp.dot(h_ref[...], ws_ref[...], preferred_element_type=jnp.float32)
    h1 = h1 + jnp.dot(mean, wn_ref[...], preferred_element_type=jnp.float32)
    h1_ref[...] = jnp.maximum(h1, 0.0)
    dinv_ref[...] = dinv


def _sage2_body(h_ref, p0_ref, p1_ref, dinv_ref, ws_ref, wn_ref,
                wp_ref, bp_ref, we_ref, be_ref, out_ref, h3_ref):
    mean = (p0_ref[...] + p1_ref[...]) * dinv_ref[...]
    h2 = jnp.dot(h_ref[...], ws_ref[...], preferred_element_type=jnp.float32)
    h2 = h2 + jnp.dot(mean, wn_ref[...], preferred_element_type=jnp.float32)
    h2 = jnp.maximum(h2, 0.0)
    out_ref[...] = jnp.dot(h2, wp_ref[...],
                           preferred_element_type=jnp.float32) + bp_ref[...]
    h3_ref[...] = jnp.dot(h2, we_ref[...],
                          preferred_element_type=jnp.float32) + be_ref[...]


def _sage3_body(h_ref, p0_ref, p1_ref, dinv_ref, ws_ref, wn_ref,
                wr_ref, br_ref, rec_ref):
    mean = (p0_ref[...] + p1_ref[...]) * dinv_ref[...]
    h = jnp.dot(h_ref[...], ws_ref[...], preferred_element_type=jnp.float32)
    h = h + jnp.dot(mean, wn_ref[...], preferred_element_type=jnp.float32)
    h = jnp.maximum(h, 0.0)
    rec_ref[...] = jnp.dot(h, wr_ref[...],
                           preferred_element_type=jnp.float32) + br_ref[...]


def kernel(x, time, n_id, edge_index, W_enc, b_enc, W_time, emb,
           W1_self, W1_nbr, W2_self, W2_nbr, Wd_self, Wd_nbr,
           W_e2d, b_e2d, W_pred, b_pred, W_recon, b_recon):
    f32 = jnp.float32
    src = edge_index[0].astype(jnp.int32)
    dst = edge_index[1].astype(jnp.int32)
    src3 = jnp.concatenate(
        [src, jnp.zeros((EPAD - E,), jnp.int32)]).reshape(NW, NCHUNK, CHUNK)
    dst3 = jnp.concatenate(
        [dst, jnp.full((EPAD - E,), TRASH, jnp.int32)]).reshape(NW, NCHUNK, CHUNK)
    nid3 = jnp.concatenate(
        [n_id.astype(jnp.int32), jnp.zeros((NID_PAD - N,), jnp.int32)]
    ).reshape(NW, GPT // GCHUNK, GCHUNK)
    z144 = jnp.zeros((CHUNK, 144), f32)
    z128 = jnp.zeros((CHUNK, 128), f32)

    emb_rows = _emb_gather()(emb, nid3)[:N]

    h0aug = _tc_call(
        _encode_body,
        [_row_spec(D), _row_spec(1), _row_spec(C), _full_spec(D, C),
         _full_spec(1, C), _full_spec(1, C)],
        _row_spec(144),
        jax.ShapeDtypeStruct((N, 144), f32),
    )(x, time[:, None], emb_rows, W_enc, b_enc[None], W_time[None])

    parts1 = _edge_pass(144)(h0aug, src3, dst3, z144)
    h0 = h0aug[:, :128]
    h1, dinv = _tc_call(
        _sage1_body,
        [_row_spec(C), _row_spec(144), _row_spec(144),
         _full_spec(C, C), _full_spec(C, C)],
        [_row_spec(C), _row_spec(1)],
        [jax.ShapeDtypeStruct((N, C), f32), jax.ShapeDtypeStruct((N, 1), f32)],
    )(h0, parts1[0, :N], parts1[1, :N], W1_self, W1_nbr)

    parts2 = _edge_pass(128)(h1, src3, dst3, z128)
    out, h3 = _tc_call(
        _sage2_body,
        [_row_spec(C), _row_spec(C), _row_spec(C), _row_spec(1),
         _full_spec(C, C), _full_spec(C, C), _full_spec(C, OUT),
         _full_spec(1, OUT), _full_spec(C, C), _full_spec(1, C)],
        [_row_spec(OUT), _row_spec(C)],
        [jax.ShapeDtypeStruct((N, OUT), f32), jax.ShapeDtypeStruct((N, C), f32)],
    )(h1, parts2[0, :N], parts2[1, :N], dinv, W2_self, W2_nbr,
      W_pred, b_pred[None], W_e2d, b_e2d[None])

    parts3 = _edge_pass(128)(h3, src3, dst3, z128)
    recon = _tc_call(
        _sage3_body,
        [_row_spec(C), _row_spec(C), _row_spec(C), _row_spec(1),
         _full_spec(C, C), _full_spec(C, C), _full_spec(C, D),
         _full_spec(1, D)],
        _row_spec(D),
        jax.ShapeDtypeStruct((N, D), f32),
    )(h3, parts3[0, :N], parts3[1, :N], dinv, Wd_self, Wd_nbr,
      W_recon, b_recon[None])

    return (out, recon)


# speedup vs baseline: 3.7055x; 3.7055x over previous
import functools

import jax
import jax.numpy as jnp
from jax import lax
from jax.experimental import pallas as pl
from jax.experimental.pallas import tpu as pltpu
from jax.experimental.pallas import tpu_sc as plsc

N = 10000
E = 320000
D = 128
C = 128
OUT = 16

NW = 32
CHUNK = 128
NCHUNK = 80
PAGE = 8
NPAGE = NCHUNK // PAGE
NC0 = 144
NC1 = 16
NP0 = NC0 // PAGE
NP1 = NC1 // PAGE
EPT = CHUNK * NCHUNK
EPAD = EPT * NW
R = 10240
RPT = R // 16
TRASH = N

NID_PAD = 10240
GCHUNK = 80
GPT = NID_PAD // NW


def _mesh():
    return plsc.VectorSubcoreMesh(core_axis_name="c", subcore_axis_name="s")


@functools.lru_cache(maxsize=None)
def _edge_pass():
    @functools.partial(
        pl.kernel,
        out_type=jax.ShapeDtypeStruct((2, R, C), jnp.float32),
        mesh=_mesh(),
        scratch_types=[
            pltpu.VMEM((2 * PAGE, CHUNK), jnp.int32),
            pltpu.VMEM((2 * PAGE, CHUNK), jnp.int32),
            pltpu.VMEM((2, CHUNK), jnp.int32),
            pltpu.VMEM((2 * CHUNK, C), jnp.float32),
            pltpu.VMEM_SHARED((R, C), jnp.float32),
            pltpu.SemaphoreType.DMA,
            pltpu.SemaphoreType.DMA,
        ],
        name="sage_edge_pass",
    )
    def kern(table, src2, dst2, zrow, parts, src_v, dst_v, meta_v, rows_v,
             acc, gsem, psem):
        c = lax.axis_index("c")
        s = lax.axis_index("s")

        pltpu.sync_copy(zrow, rows_v.at[pl.ds(0, CHUNK)])
        for k in range(RPT // CHUNK):
            pltpu.sync_copy(rows_v.at[pl.ds(0, CHUNK)],
                            acc.at[pl.ds(s * RPT + k * CHUNK, CHUNK)])
        plsc.subcore_barrier()

        base = jnp.where(c == 0, s * NC0, 16 * NC0 + s * NC1)
        np_ = jnp.where(c == 0, NP0, NP1)
        for k in range(2 * CHUNK // 16):
            meta_v[k // 8, pl.ds((k % 8) * 16, 16)] = lax.iota(
                jnp.int32, 16) + (base + k * 16)

        def fetch_page(p, half):
            e0 = p * PAGE
            ids = meta_v.at[e0 // CHUNK, pl.ds(e0 % CHUNK, PAGE)]
            dst_sl = pl.ds(half * PAGE, PAGE)
            pltpu.async_copy(src2.at[ids], src_v.at[dst_sl], psem)
            pltpu.async_copy(dst2.at[ids], dst_v.at[dst_sl], psem)

        def wait_page():
            for _ in range(2):
                pltpu.make_async_copy(
                    src2.at[meta_v.at[0, pl.ds(0, PAGE)]],
                    src_v.at[pl.ds(0, PAGE)], psem).wait()

        def issue_gather(row, off):
            pltpu.async_copy(table.at[src_v.at[row, pl.ds(0, CHUNK // 2)]],
                             rows_v.at[pl.ds(off, CHUNK // 2)], gsem)
            pltpu.async_copy(
                table.at[src_v.at[row, pl.ds(CHUNK // 2, CHUNK // 2)]],
                rows_v.at[pl.ds(off + CHUNK // 2, CHUNK // 2)], gsem)

        def wait_gather():
            for _ in range(2):
                pltpu.make_async_copy(
                    table.at[src_v.at[0, pl.ds(0, CHUNK // 2)]],
                    rows_v.at[pl.ds(0, CHUNK // 2)], gsem).wait()

        @pl.when(np_ > 0)
        def _():
            fetch_page(0, 0)
            wait_page()
            issue_gather(0, 0)

        def pbody(p, carry):
            pb = lax.rem(p, 2)

            @pl.when(p < np_ - 1)
            def _():
                fetch_page(p + 1, 1 - pb)

            for r in range(PAGE):
                off = (r % 2) * CHUNK
                wait_gather()
                if r < PAGE - 1:
                    issue_gather(pb * PAGE + r + 1, (1 - r % 2) * CHUNK)
                else:
                    @pl.when(p < np_ - 1)
                    def _():
                        wait_page()
                        issue_gather((1 - pb) * PAGE, (1 - r % 2) * CHUNK)
                pltpu.sync_copy(rows_v.at[pl.ds(off, CHUNK)],
                                acc.at[dst_v.at[pb * PAGE + r]], add=True)
            return carry

        lax.fori_loop(0, np_, pbody, 0)
        plsc.subcore_barrier()

        for k in range(RPT // CHUNK):
            rbase = s * RPT + k * CHUNK
            pltpu.sync_copy(acc.at[pl.ds(rbase, CHUNK)],
                            rows_v.at[pl.ds(0, CHUNK)])
            pltpu.sync_copy(rows_v.at[pl.ds(0, CHUNK)],
                            parts.at[c, pl.ds(rbase, CHUNK)])

    return kern


@functools.lru_cache(maxsize=None)
def _deg_pass():
    @functools.partial(
        pl.kernel,
        out_type=jax.ShapeDtypeStruct((2, R, C), jnp.float32),
        mesh=_mesh(),
        scratch_types=[
            pltpu.VMEM((2 * PAGE, CHUNK), jnp.int32),
            pltpu.VMEM((1, NCHUNK), jnp.int32),
            pltpu.VMEM((CHUNK, C), jnp.float32),
            pltpu.VMEM((CHUNK, C), jnp.float32),
            pltpu.VMEM_SHARED((R, C), jnp.float32),
            pltpu.SemaphoreType.DMA,
        ],
        name="deg_pass",
    )
    def kern(dst2, ones, zrow, parts, dst_v, meta_v, ones_v, stage_v, acc,
             psem):
        c = lax.axis_index("c")
        s = lax.axis_index("s")
        wid = s * 2 + c

        pltpu.sync_copy(zrow, stage_v)
        for k in range(RPT // CHUNK):
            pltpu.sync_copy(stage_v, acc.at[pl.ds(s * RPT + k * CHUNK, CHUNK)])
        pltpu.sync_copy(ones, ones_v)
        base = wid * NCHUNK
        for k in range(NCHUNK // 16):
            meta_v[0, pl.ds(k * 16, 16)] = lax.iota(jnp.int32, 16) + (
                base + k * 16)

        def fetch_page(p, half):
            pltpu.async_copy(dst2.at[meta_v.at[0, pl.ds(p * PAGE, PAGE)]],
                             dst_v.at[pl.ds(half * PAGE, PAGE)], psem)

        def wait_page():
            pltpu.make_async_copy(
                dst2.at[meta_v.at[0, pl.ds(0, PAGE)]],
                dst_v.at[pl.ds(0, PAGE)], psem).wait()

        fetch_page(0, 0)
        wait_page()
        plsc.subcore_barrier()

        def pbody(p, carry):
            pb = lax.rem(p, 2)

            @pl.when(p < NPAGE - 1)
            def _():
                fetch_page(p + 1, 1 - pb)

            for r in range(PAGE):
                pltpu.sync_copy(ones_v, acc.at[dst_v.at[pb * PAGE + r]],
                                add=True)

            @pl.when(p < NPAGE - 1)
            def _():
                wait_page()
            return carry

        lax.fori_loop(0, NPAGE, pbody, 0)
        plsc.subcore_barrier()

        for k in range(RPT // CHUNK):
            base = s * RPT + k * CHUNK
            pltpu.sync_copy(acc.at[pl.ds(base, CHUNK)], stage_v)
            pltpu.sync_copy(stage_v, parts.at[c, pl.ds(base, CHUNK)])

    return kern


@functools.lru_cache(maxsize=None)
def _emb_gather():
    @functools.partial(
        pl.kernel,
        out_type=jax.ShapeDtypeStruct((NID_PAD, C), jnp.float32),
        mesh=_mesh(),
        scratch_types=[
            pltpu.VMEM((GPT // GCHUNK, GCHUNK), jnp.int32),
            pltpu.VMEM((GCHUNK, C), jnp.float32),
            pltpu.SemaphoreType.DMA,
        ],
        name="emb_gather",
    )
    def kern(table, idx3, out, idx_v, rows_v, gsem):
        c = lax.axis_index("c")
        s = lax.axis_index("s")
        wid = s * 2 + c
        base = wid * GPT
        pltpu.sync_copy(idx3.at[wid], idx_v)
        for j in range(GPT // GCHUNK):
            pltpu.async_copy(table.at[idx_v.at[j]], rows_v, gsem).wait()
            pltpu.sync_copy(rows_v, out.at[pl.ds(base + j * GCHUNK, GCHUNK)])

    return kern


BM = 2000
GRID = N // BM


def _row_spec(w):
    return pl.BlockSpec((BM, w), lambda i: (i, 0))


def _full_spec(a, b):
    return pl.BlockSpec((a, b), lambda i: (0, 0))


def _tc_call(body, in_specs, out_specs, out_shapes):
    return pl.pallas_call(
        body,
        grid=(GRID,),
        in_specs=in_specs,
        out_specs=out_specs,
        out_shape=out_shapes,
    )


def _encode_body(x_ref, t_ref, er_ref, w_ref, b_ref, wt_ref, o_ref):
    h = jnp.dot(x_ref[...], w_ref[...], preferred_element_type=jnp.float32)
    o_ref[...] = h + b_ref[...] + t_ref[...] * wt_ref[...] + er_ref[...]


def _sage1_body(h_ref, p0_ref, p1_ref, d0_ref, d1_ref, ws_ref, wn_ref,
                h1_ref, dinv_ref):
    p = p0_ref[...] + p1_ref[...]
    deg = d0_ref[:, 0:1] + d1_ref[:, 0:1]
    dinv = 1.0 / jnp.maximum(deg, 1.0)
    mean = p * dinv
    h1 = jnp.dot(h_ref[...], ws_ref[...], preferred_element_type=jnp.float32)
    h1 = h1 + jnp.dot(mean, wn_ref[...], preferred_element_type=jnp.float32)
    h1_ref[...] = jnp.maximum(h1, 0.0)
    dinv_ref[...] = dinv


def _sage2_body(h_ref, p0_ref, p1_ref, dinv_ref, ws_ref, wn_ref,
                wp_ref, bp_ref, we_ref, be_ref, out_ref, h3_ref):
    mean = (p0_ref[...] + p1_ref[...]) * dinv_ref[...]
    h2 = jnp.dot(h_ref[...], ws_ref[...], preferred_element_type=jnp.float32)
    h2 = h2 + jnp.dot(mean, wn_ref[...], preferred_element_type=jnp.float32)
    h2 = jnp.maximum(h2, 0.0)
    out_ref[...] = jnp.dot(h2, wp_ref[...],
                           preferred_element_type=jnp.float32) + bp_ref[...]
    h3_ref[...] = jnp.dot(h2, we_ref[...],
                          preferred_element_type=jnp.float32) + be_ref[...]


def _sage3_body(h_ref, p0_ref, p1_ref, dinv_ref, ws_ref, wn_ref,
                wr_ref, br_ref, rec_ref):
    mean = (p0_ref[...] + p1_ref[...]) * dinv_ref[...]
    h = jnp.dot(h_ref[...], ws_ref[...], preferred_element_type=jnp.float32)
    h = h + jnp.dot(mean, wn_ref[...], preferred_element_type=jnp.float32)
    h = jnp.maximum(h, 0.0)
    rec_ref[...] = jnp.dot(h, wr_ref[...],
                           preferred_element_type=jnp.float32) + br_ref[...]


def kernel(x, time, n_id, edge_index, W_enc, b_enc, W_time, emb,
           W1_self, W1_nbr, W2_self, W2_nbr, Wd_self, Wd_nbr,
           W_e2d, b_e2d, W_pred, b_pred, W_recon, b_recon):
    f32 = jnp.float32
    src = edge_index[0].astype(jnp.int32)
    dst = edge_index[1].astype(jnp.int32)
    src2 = jnp.concatenate(
        [src, jnp.zeros((EPAD - E,), jnp.int32)]).reshape(NW * NCHUNK, CHUNK)
    dst2 = jnp.concatenate(
        [dst, jnp.full((EPAD - E,), TRASH, jnp.int32)]).reshape(NW * NCHUNK, CHUNK)
    nid3 = jnp.concatenate(
        [n_id.astype(jnp.int32), jnp.zeros((NID_PAD - N,), jnp.int32)]
    ).reshape(NW, GPT // GCHUNK, GCHUNK)
    z128 = jnp.zeros((CHUNK, 128), f32)
    ones128 = jnp.ones((CHUNK, 128), f32)

    emb_rows = _emb_gather()(emb, nid3)[:N]
    deg_parts = _deg_pass()(dst2, ones128, z128)

    h0 = _tc_call(
        _encode_body,
        [_row_spec(D), _row_spec(1), _row_spec(C), _full_spec(D, C),
         _full_spec(1, C), _full_spec(1, C)],
        _row_spec(C),
        jax.ShapeDtypeStruct((N, C), f32),
    )(x, time[:, None], emb_rows, W_enc, b_enc[None], W_time[None])

    parts1 = _edge_pass()(h0, src2, dst2, z128)
    h1, dinv = _tc_call(
        _sage1_body,
        [_row_spec(C), _row_spec(C), _row_spec(C), _row_spec(C), _row_spec(C),
         _full_spec(C, C), _full_spec(C, C)],
        [_row_spec(C), _row_spec(1)],
        [jax.ShapeDtypeStruct((N, C), f32), jax.ShapeDtypeStruct((N, 1), f32)],
    )(h0, parts1[0, :N], parts1[1, :N], deg_parts[0, :N], deg_parts[1, :N],
      W1_self, W1_nbr)

    parts2 = _edge_pass()(h1, src2, dst2, z128)
    out, h3 = _tc_call(
        _sage2_body,
        [_row_spec(C), _row_spec(C), _row_spec(C), _row_spec(1),
         _full_spec(C, C), _full_spec(C, C), _full_spec(C, OUT),
         _full_spec(1, OUT), _full_spec(C, C), _full_spec(1, C)],
        [_row_spec(OUT), _row_spec(C)],
        [jax.ShapeDtypeStruct((N, OUT), f32), jax.ShapeDtypeStruct((N, C), f32)],
    )(h1, parts2[0, :N], parts2[1, :N], dinv, W2_self, W2_nbr,
      W_pred, b_pred[None], W_e2d, b_e2d[None])

    parts3 = _edge_pass()(h3, src2, dst2, z128)
    recon = _tc_call(
        _sage3_body,
        [_row_spec(C), _row_spec(C), _row_spec(C), _row_spec(1),
         _full_spec(C, C), _full_spec(C, C), _full_spec(C, D),
         _full_spec(1, D)],
        _row_spec(D),
        jax.ShapeDtypeStruct((N, D), f32),
    )(h3, parts3[0, :N], parts3[1, :N], dinv, Wd_self, Wd_nbr,
      W_recon, b_recon[None])

    return (out, recon)
